# baseline (speedup 1.0000x reference)
import numpy as np
import jax
import jax.numpy as jnp
from jax import lax
from jax.experimental import pallas as pl
from jax.experimental.pallas import tpu as pltpu

T = 20
SEED = 42
KMAX = 64
KCHUNK = 8
RCHUNK = 2
RMAX = 32
LANES = 128

_ROT0 = (13, 15, 26, 6)
_ROT1 = (17, 29, 16, 24)
_KS_PARITY = 0x1BD11BDA


def _np_tf_block(k1, k2, x0, x1):
    k1 = np.uint32(k1); k2 = np.uint32(k2)
    ks = [k1, k2, np.uint32(k1 ^ k2 ^ np.uint32(_KS_PARITY))]
    v0 = (x0.astype(np.uint32) + ks[0]).astype(np.uint32)
    v1 = (x1.astype(np.uint32) + ks[1]).astype(np.uint32)

    def rnd(v0, v1, rots):
        for r in rots:
            v0 = (v0 + v1).astype(np.uint32)
            v1 = ((v1 << np.uint32(r)) | (v1 >> np.uint32(32 - r))).astype(np.uint32)
            v1 = v1 ^ v0
        return v0, v1

    for i, (rots, a, b) in enumerate(
            [(_ROT0, 1, 2), (_ROT1, 2, 0), (_ROT0, 0, 1), (_ROT1, 1, 2), (_ROT0, 2, 0)]):
        v0, v1 = rnd(v0, v1, rots)
        v0 = (v0 + ks[a]).astype(np.uint32)
        v1 = (v1 + ks[b] + np.uint32(i + 1)).astype(np.uint32)
    return v0, v1


def _subkey_chains():
    key = (np.uint32(0), np.uint32(SEED))
    rng = key
    ksubs = np.zeros((KMAX, 2), np.uint32)
    for i in range(KMAX):
        w0, w1 = _np_tf_block(rng[0], rng[1], np.zeros(2, np.uint32),
                              np.arange(2, dtype=np.uint32))
        rng = (w0[0], w1[0])
        ksubs[i] = (w0[1], w1[1])
    k = key
    rsubs = np.zeros((RMAX, 2, 2), np.uint32)
    for i in range(RMAX):
        w0, w1 = _np_tf_block(k[0], k[1], np.zeros(3, np.uint32),
                              np.arange(3, dtype=np.uint32))
        k = (w0[0], w1[0])
        rsubs[i, 0] = (w0[1], w1[1])
        rsubs[i, 1] = (w0[2], w1[2])
    return ksubs, rsubs


_KSUBS, _RSUBS = _subkey_chains()


def _tf_bits(k1, k2, x1):
    ks2 = k1 ^ k2 ^ jnp.uint32(_KS_PARITY)
    ks = (k1, k2, ks2)
    v0 = jnp.zeros_like(x1) + k1
    v1 = x1 + k2

    def rnd(v0, v1, rots):
        for r in rots:
            v0 = v0 + v1
            v1 = (v1 << jnp.uint32(r)) | (v1 >> jnp.uint32(32 - r))
            v1 = v1 ^ v0
        return v0, v1

    for i, (rots, a, b) in enumerate(
            [(_ROT0, 1, 2), (_ROT1, 2, 0), (_ROT0, 0, 1), (_ROT1, 1, 2), (_ROT0, 2, 0)]):
        v0, v1 = rnd(v0, v1, rots)
        v0 = v0 + ks[a]
        v1 = v1 + ks[b] + jnp.uint32(i + 1)
    return v0 ^ v1


def _u01(bits):
    fb = (bits >> jnp.uint32(9)) | jnp.uint32(0x3F800000)
    return lax.bitcast_convert_type(fb, jnp.float32) - jnp.float32(1.0)


def _lgamma(x):
    kBase = np.float32(0.99999999999980993227684700473478)
    coeffs = (676.520368121885098567009190444019,
              -1259.13921672240287047156078755283,
              771.3234287776530788486528258894,
              -176.61502916214059906584551354,
              12.507343278686904814458936853,
              -0.13857109526572011689554707,
              9.984369578019570859563e-6,
              1.50563273514931155834e-7)
    z = x - jnp.float32(1.0)
    acc = jnp.zeros_like(z) + kBase
    for i, c in enumerate(coeffs):
        acc = acc + np.float32(c) / (z + np.float32(i + 1))
    g_half = np.float32(7.5)
    t = g_half + z
    log_t = np.float32(2.0149030205422647) + lax.log1p(z / g_half)
    log_sqrt_two_pi = np.float32(0.91893853320467274178032973640562)
    return log_sqrt_two_pi + (z + np.float32(0.5) - t / log_t) * log_t + jnp.log(acc)


def _rej_consts(lam_r):
    log_lam = jnp.log(lam_r)
    b = jnp.float32(0.931) + jnp.float32(2.53) * jnp.sqrt(lam_r)
    a = jnp.float32(-0.059) + jnp.float32(0.02483) * b
    inv_alpha = jnp.float32(1.1239) + jnp.float32(1.1328) / (b - jnp.float32(3.4))
    v_r = jnp.float32(0.9277) - jnp.float32(3.6224) / (b - jnp.float32(2))
    return log_lam, b, a, inv_alpha, v_r


def _rej_draw(u, v, lam_r, log_lam, a, b, inv_alpha, v_r):
    u_shifted = jnp.float32(0.5) - jnp.abs(u)
    k = jnp.floor((jnp.float32(2) * a / u_shifted + b) * u + lam_r + jnp.float32(0.43))
    s = jnp.log(v * inv_alpha / (a / (u_shifted * u_shifted) + b))
    t = -lam_r + k * log_lam - _lgamma(k + jnp.float32(1))
    accept1 = (u_shifted >= jnp.float32(0.07)) & (v <= v_r)
    reject = (k < 0) | ((u_shifted < jnp.float32(0.013)) & (v > u_shifted))
    accept2 = s <= t
    return k, accept1 | (~reject & accept2)


def _lam_tiles(flat):
    nonzero = flat != jnp.float32(0.0)
    safe = jnp.where(nonzero, flat, jnp.float32(1.0))
    lam = jnp.where(nonzero, jnp.float32(1.0) / safe, jnp.float32(0.0))
    use_knuth = lam < jnp.float32(10.0)
    lam_r = jnp.where(use_knuth, jnp.float32(1e5), lam)
    return nonzero, lam, use_knuth, lam_r


def _pix_counters(pid, s):
    r_i = lax.broadcasted_iota(jnp.int32, (s, LANES), 0)
    c_i = lax.broadcasted_iota(jnp.int32, (s, LANES), 1)
    return (pid * s + r_i) * LANES + c_i


def _make_pass_a(s, n_pix):
    def body(ksub_ref, rsub_ref, img_ref, ival_ref, m_ref, n_ref):
        pid = pl.program_id(0)
        flat = img_ref[...]
        _, lam, use_knuth, lam_r = _lam_tiles(flat)
        neg_lam_k = -jnp.where(use_knuth, lam, jnp.float32(0.0))
        log_lam, b, a, inv_alpha, v_r = _rej_consts(lam_r)
        pix = _pix_counters(pid, s)

        m_blk = jnp.int32(0)
        for t in range(T):
            j = (pix + t * n_pix).astype(jnp.uint32)

            def kcond(carry):
                i, _, lp = carry
                return (i < KMAX) & jnp.any(lp > neg_lam_k)

            def kloop(carry):
                i, k, lp = carry
                for cc in range(KCHUNK):
                    k = jnp.where(lp > neg_lam_k, k + jnp.float32(1.0), k)
                    u = _u01(_tf_bits(ksub_ref[i + cc, 0], ksub_ref[i + cc, 1], j))
                    lp = lp + jnp.log(u)
                return i + KCHUNK, k, lp

            _, kk, _ = lax.while_loop(
                kcond, kloop,
                (jnp.int32(0), jnp.zeros((s, LANES), jnp.float32),
                 jnp.zeros((s, LANES), jnp.float32)))
            k_knuth = kk - jnp.float32(1.0)

            def rcond(carry):
                i, _, active, _ = carry
                return (i < RMAX) & (jnp.max(active) > jnp.float32(0.0))

            def rloop(carry):
                i, k_out, active, mvec = carry
                for cc in range(RCHUNK):
                    u = _u01(_tf_bits(rsub_ref[i + cc, 0, 0], rsub_ref[i + cc, 0, 1], j)) - jnp.float32(0.5)
                    v = _u01(_tf_bits(rsub_ref[i + cc, 1, 0], rsub_ref[i + cc, 1, 1], j))
                    k, accept = _rej_draw(u, v, lam_r, log_lam, a, b, inv_alpha, v_r)
                    g = jnp.max(active, axis=(0, 1), keepdims=True)
                    live = g > jnp.float32(0.0)
                    k_out = jnp.where(accept & live, k, k_out)
                    mvec = mvec + jnp.where(live, jnp.float32(1.0), jnp.float32(0.0))
                    active = jnp.where(accept, jnp.float32(0.0), active)
                return i + RCHUNK, k_out, active, mvec

            _, k_rej, _, mvec = lax.while_loop(
                rcond, rloop,
                (jnp.int32(0), jnp.full((s, LANES), -1.0, jnp.float32),
                 jnp.ones((s, LANES), jnp.float32),
                 jnp.zeros((1, 1), jnp.float32)))
            m_t = jnp.max(mvec).astype(jnp.int32)

            ival_ref[t] = jnp.where(use_knuth, k_knuth, k_rej)
            m_ref[pid, t] = m_t
            m_blk = jnp.maximum(m_blk, m_t)

        @pl.when(pid == 0)
        def _():
            n_ref[0, 0] = jnp.int32(0)

        n_ref[0, 0] = jnp.maximum(n_ref[0, 0], m_blk)

    return body


def _make_pass_b(s, n_pix):
    def body(rsub_ref, n_ref, m_ref, img_ref, ival_in_ref, out_ref):
        pid = pl.program_id(0)
        flat = img_ref[...]
        nonzero, lam, use_knuth, lam_r = _lam_tiles(flat)
        log_lam, b, a, inv_alpha, v_r = _rej_consts(lam_r)
        pix = _pix_counters(pid, s)
        n_iters = n_ref[0, 0]

        c = jnp.zeros((s, LANES), jnp.float32)
        cs = []
        for t in range(T):
            j = (pix + t * n_pix).astype(jnp.uint32)

            def rloop(carry):
                i, k_out = carry
                u = _u01(_tf_bits(rsub_ref[i, 0, 0], rsub_ref[i, 0, 1], j)) - jnp.float32(0.5)
                v = _u01(_tf_bits(rsub_ref[i, 1, 0], rsub_ref[i, 1, 1], j))
                k, accept = _rej_draw(u, v, lam_r, log_lam, a, b, inv_alpha, v_r)
                return i + 1, jnp.where(accept & ~use_knuth, k, k_out)

            _, ival = lax.while_loop(
                lambda cr: cr[0] < n_iters, rloop,
                (m_ref[pid, t], ival_in_ref[t]))

            ival = jnp.where(lam == jnp.float32(0.0), jnp.float32(0.0), ival)
            ival = jnp.where(nonzero & (ival == jnp.float32(0.0)), jnp.float32(1.0), ival)
            c = c + ival
            cs.append(c)

        for r in range(T):
            tv = jnp.float32(r + 1)
            acc = cs[0] == tv
            for t in range(1, r + 1):
                acc = acc | (cs[t] == tv)
            out_ref[r] = acc

    return body


def kernel(img):
    shape = img.shape
    n_pix = int(np.prod(shape))
    assert n_pix % LANES == 0
    rows = n_pix // LANES
    s = 128 if rows % 128 == 0 else rows
    grid = (rows // s,)
    nblk = rows // s
    flat2d = img.reshape(rows, LANES)

    ksubs = jnp.asarray(_KSUBS)
    rsubs = jnp.asarray(_RSUBS)

    ival_a, m_arr, n_arr = pl.pallas_call(
        _make_pass_a(s, n_pix),
        grid=grid,
        in_specs=[
            pl.BlockSpec(memory_space=pltpu.SMEM),
            pl.BlockSpec(memory_space=pltpu.SMEM),
            pl.BlockSpec((s, LANES), lambda i: (i, 0)),
        ],
        out_specs=[
            pl.BlockSpec((T, s, LANES), lambda i: (0, i, 0)),
            pl.BlockSpec(memory_space=pltpu.SMEM),
            pl.BlockSpec(memory_space=pltpu.SMEM),
        ],
        out_shape=[
            jax.ShapeDtypeStruct((T, rows, LANES), jnp.float32),
            jax.ShapeDtypeStruct((nblk, T), jnp.int32),
            jax.ShapeDtypeStruct((1, 1), jnp.int32),
        ],
        compiler_params=pltpu.CompilerParams(
            dimension_semantics=("arbitrary",)),
    )(ksubs, rsubs, flat2d)

    spikes = pl.pallas_call(
        _make_pass_b(s, n_pix),
        grid=grid,
        in_specs=[
            pl.BlockSpec(memory_space=pltpu.SMEM),
            pl.BlockSpec(memory_space=pltpu.SMEM),
            pl.BlockSpec(memory_space=pltpu.SMEM),
            pl.BlockSpec((s, LANES), lambda i: (i, 0)),
            pl.BlockSpec((T, s, LANES), lambda i: (0, i, 0)),
        ],
        out_specs=pl.BlockSpec((T, s, LANES), lambda i: (0, i, 0)),
        out_shape=jax.ShapeDtypeStruct((T, rows, LANES), jnp.bool_),
        compiler_params=pltpu.CompilerParams(
            dimension_semantics=("arbitrary",)),
    )(rsubs, n_arr, m_arr, flat2d, ival_a)

    return spikes.reshape((T,) + tuple(shape))


# speedup vs baseline: 1.0045x; 1.0045x over previous
import numpy as np
import jax
import jax.numpy as jnp
from jax import lax
from jax.experimental import pallas as pl
from jax.experimental.pallas import tpu as pltpu

T = 20
SEED = 42
KMAX = 64
KCHUNK = 4
RCHUNK = 2
RMAX = 32
LANES = 128

_ROT0 = (13, 15, 26, 6)
_ROT1 = (17, 29, 16, 24)
_KS_PARITY = 0x1BD11BDA


def _np_tf_block(k1, k2, x0, x1):
    k1 = np.uint32(k1); k2 = np.uint32(k2)
    ks = [k1, k2, np.uint32(k1 ^ k2 ^ np.uint32(_KS_PARITY))]
    v0 = (x0.astype(np.uint32) + ks[0]).astype(np.uint32)
    v1 = (x1.astype(np.uint32) + ks[1]).astype(np.uint32)

    def rnd(v0, v1, rots):
        for r in rots:
            v0 = (v0 + v1).astype(np.uint32)
            v1 = ((v1 << np.uint32(r)) | (v1 >> np.uint32(32 - r))).astype(np.uint32)
            v1 = v1 ^ v0
        return v0, v1

    for i, (rots, a, b) in enumerate(
            [(_ROT0, 1, 2), (_ROT1, 2, 0), (_ROT0, 0, 1), (_ROT1, 1, 2), (_ROT0, 2, 0)]):
        v0, v1 = rnd(v0, v1, rots)
        v0 = (v0 + ks[a]).astype(np.uint32)
        v1 = (v1 + ks[b] + np.uint32(i + 1)).astype(np.uint32)
    return v0, v1


def _subkey_chains():
    key = (np.uint32(0), np.uint32(SEED))
    rng = key
    ksubs = np.zeros((KMAX, 2), np.uint32)
    for i in range(KMAX):
        w0, w1 = _np_tf_block(rng[0], rng[1], np.zeros(2, np.uint32),
                              np.arange(2, dtype=np.uint32))
        rng = (w0[0], w1[0])
        ksubs[i] = (w0[1], w1[1])
    k = key
    rsubs = np.zeros((RMAX, 2, 2), np.uint32)
    for i in range(RMAX):
        w0, w1 = _np_tf_block(k[0], k[1], np.zeros(3, np.uint32),
                              np.arange(3, dtype=np.uint32))
        k = (w0[0], w1[0])
        rsubs[i, 0] = (w0[1], w1[1])
        rsubs[i, 1] = (w0[2], w1[2])
    return ksubs, rsubs


_KSUBS, _RSUBS = _subkey_chains()


def _tf_bits(k1, k2, x1):
    ks2 = k1 ^ k2 ^ jnp.uint32(_KS_PARITY)
    ks = (k1, k2, ks2)
    v0 = jnp.zeros_like(x1) + k1
    v1 = x1 + k2

    def rnd(v0, v1, rots):
        for r in rots:
            v0 = v0 + v1
            v1 = (v1 << jnp.uint32(r)) | (v1 >> jnp.uint32(32 - r))
            v1 = v1 ^ v0
        return v0, v1

    for i, (rots, a, b) in enumerate(
            [(_ROT0, 1, 2), (_ROT1, 2, 0), (_ROT0, 0, 1), (_ROT1, 1, 2), (_ROT0, 2, 0)]):
        v0, v1 = rnd(v0, v1, rots)
        v0 = v0 + ks[a]
        v1 = v1 + ks[b] + jnp.uint32(i + 1)
    return v0 ^ v1


def _u01(bits):
    fb = (bits >> jnp.uint32(9)) | jnp.uint32(0x3F800000)
    return lax.bitcast_convert_type(fb, jnp.float32) - jnp.float32(1.0)


def _lgamma(x):
    kBase = np.float32(0.99999999999980993227684700473478)
    coeffs = (676.520368121885098567009190444019,
              -1259.13921672240287047156078755283,
              771.3234287776530788486528258894,
              -176.61502916214059906584551354,
              12.507343278686904814458936853,
              -0.13857109526572011689554707,
              9.984369578019570859563e-6,
              1.50563273514931155834e-7)
    z = x - jnp.float32(1.0)
    acc = jnp.zeros_like(z) + kBase
    for i, c in enumerate(coeffs):
        acc = acc + np.float32(c) / (z + np.float32(i + 1))
    g_half = np.float32(7.5)
    t = g_half + z
    log_t = np.float32(2.0149030205422647) + lax.log1p(z / g_half)
    log_sqrt_two_pi = np.float32(0.91893853320467274178032973640562)
    return log_sqrt_two_pi + (z + np.float32(0.5) - t / log_t) * log_t + jnp.log(acc)


def _rej_consts(lam_r):
    log_lam = jnp.log(lam_r)
    b = jnp.float32(0.931) + jnp.float32(2.53) * jnp.sqrt(lam_r)
    a = jnp.float32(-0.059) + jnp.float32(0.02483) * b
    inv_alpha = jnp.float32(1.1239) + jnp.float32(1.1328) / (b - jnp.float32(3.4))
    v_r = jnp.float32(0.9277) - jnp.float32(3.6224) / (b - jnp.float32(2))
    return log_lam, b, a, inv_alpha, v_r


def _rej_draw(u, v, lam_r, log_lam, a, b, inv_alpha, v_r):
    u_shifted = jnp.float32(0.5) - jnp.abs(u)
    k = jnp.floor((jnp.float32(2) * a / u_shifted + b) * u + lam_r + jnp.float32(0.43))
    s = jnp.log(v * inv_alpha / (a / (u_shifted * u_shifted) + b))
    t = -lam_r + k * log_lam - _lgamma(k + jnp.float32(1))
    accept1 = (u_shifted >= jnp.float32(0.07)) & (v <= v_r)
    reject = (k < 0) | ((u_shifted < jnp.float32(0.013)) & (v > u_shifted))
    accept2 = s <= t
    return k, accept1 | (~reject & accept2)


def _lam_tiles(flat):
    nonzero = flat != jnp.float32(0.0)
    safe = jnp.where(nonzero, flat, jnp.float32(1.0))
    lam = jnp.where(nonzero, jnp.float32(1.0) / safe, jnp.float32(0.0))
    use_knuth = lam < jnp.float32(10.0)
    lam_r = jnp.where(use_knuth, jnp.float32(1e5), lam)
    return nonzero, lam, use_knuth, lam_r


def _pix_counters(pid, s):
    r_i = lax.broadcasted_iota(jnp.int32, (s, LANES), 0)
    c_i = lax.broadcasted_iota(jnp.int32, (s, LANES), 1)
    return (pid * s + r_i) * LANES + c_i


def _make_pass_a(s, n_pix):
    def body(ksub_ref, rsub_ref, img_ref, ival_ref, m_ref, n_ref):
        pid = pl.program_id(0)
        flat = img_ref[...]
        _, lam, use_knuth, lam_r = _lam_tiles(flat)
        neg_lam_k = -jnp.where(use_knuth, lam, jnp.float32(0.0))
        log_lam, b, a, inv_alpha, v_r = _rej_consts(lam_r)
        pix = _pix_counters(pid, s)

        m_blk = jnp.int32(0)
        for t in range(T):
            j = (pix + t * n_pix).astype(jnp.uint32)

            def kcond(carry):
                i, _, lp = carry
                return (i < KMAX) & jnp.any(lp > neg_lam_k)

            def kloop(carry):
                i, k, lp = carry
                for cc in range(KCHUNK):
                    k = jnp.where(lp > neg_lam_k, k + jnp.float32(1.0), k)
                    u = _u01(_tf_bits(ksub_ref[i + cc, 0], ksub_ref[i + cc, 1], j))
                    lp = lp + jnp.log(u)
                return i + KCHUNK, k, lp

            _, kk, _ = lax.while_loop(
                kcond, kloop,
                (jnp.int32(0), jnp.zeros((s, LANES), jnp.float32),
                 jnp.zeros((s, LANES), jnp.float32)))
            k_knuth = kk - jnp.float32(1.0)

            def rcond(carry):
                i, _, active, _ = carry
                return (i < RMAX) & (jnp.max(active) > jnp.float32(0.0))

            def rloop(carry):
                i, k_out, active, mvec = carry
                for cc in range(RCHUNK):
                    u = _u01(_tf_bits(rsub_ref[i + cc, 0, 0], rsub_ref[i + cc, 0, 1], j)) - jnp.float32(0.5)
                    v = _u01(_tf_bits(rsub_ref[i + cc, 1, 0], rsub_ref[i + cc, 1, 1], j))
                    k, accept = _rej_draw(u, v, lam_r, log_lam, a, b, inv_alpha, v_r)
                    g = jnp.max(active, axis=(0, 1), keepdims=True)
                    live = g > jnp.float32(0.0)
                    k_out = jnp.where(accept & live, k, k_out)
                    mvec = mvec + jnp.where(live, jnp.float32(1.0), jnp.float32(0.0))
                    active = jnp.where(accept, jnp.float32(0.0), active)
                return i + RCHUNK, k_out, active, mvec

            _, k_rej, _, mvec = lax.while_loop(
                rcond, rloop,
                (jnp.int32(0), jnp.full((s, LANES), -1.0, jnp.float32),
                 jnp.ones((s, LANES), jnp.float32),
                 jnp.zeros((1, 1), jnp.float32)))
            m_t = jnp.max(mvec).astype(jnp.int32)

            ival_ref[t] = jnp.where(use_knuth, k_knuth, k_rej)
            m_ref[pid, t] = m_t
            m_blk = jnp.maximum(m_blk, m_t)

        @pl.when(pid == 0)
        def _():
            n_ref[0, 0] = jnp.int32(0)

        n_ref[0, 0] = jnp.maximum(n_ref[0, 0], m_blk)

    return body


def _make_pass_b(s, n_pix):
    def body(rsub_ref, n_ref, m_ref, img_ref, ival_in_ref, out_ref):
        pid = pl.program_id(0)
        flat = img_ref[...]
        nonzero, lam, use_knuth, lam_r = _lam_tiles(flat)
        log_lam, b, a, inv_alpha, v_r = _rej_consts(lam_r)
        pix = _pix_counters(pid, s)
        n_iters = n_ref[0, 0]

        c = jnp.zeros((s, LANES), jnp.float32)
        cs = []
        for t in range(T):
            j = (pix + t * n_pix).astype(jnp.uint32)

            def rloop(carry):
                i, k_out = carry
                u = _u01(_tf_bits(rsub_ref[i, 0, 0], rsub_ref[i, 0, 1], j)) - jnp.float32(0.5)
                v = _u01(_tf_bits(rsub_ref[i, 1, 0], rsub_ref[i, 1, 1], j))
                k, accept = _rej_draw(u, v, lam_r, log_lam, a, b, inv_alpha, v_r)
                return i + 1, jnp.where(accept & ~use_knuth, k, k_out)

            _, ival = lax.while_loop(
                lambda cr: cr[0] < n_iters, rloop,
                (m_ref[pid, t], ival_in_ref[t]))

            ival = jnp.where(lam == jnp.float32(0.0), jnp.float32(0.0), ival)
            ival = jnp.where(nonzero & (ival == jnp.float32(0.0)), jnp.float32(1.0), ival)
            c = c + ival
            cs.append(c)

        for r in range(T):
            tv = jnp.float32(r + 1)
            acc = cs[0] == tv
            for t in range(1, r + 1):
                acc = acc | (cs[t] == tv)
            out_ref[r] = acc

    return body


def kernel(img):
    shape = img.shape
    n_pix = int(np.prod(shape))
    assert n_pix % LANES == 0
    rows = n_pix // LANES
    s = 64 if rows % 64 == 0 else rows
    grid = (rows // s,)
    nblk = rows // s
    flat2d = img.reshape(rows, LANES)

    ksubs = jnp.asarray(_KSUBS)
    rsubs = jnp.asarray(_RSUBS)

    ival_a, m_arr, n_arr = pl.pallas_call(
        _make_pass_a(s, n_pix),
        grid=grid,
        in_specs=[
            pl.BlockSpec(memory_space=pltpu.SMEM),
            pl.BlockSpec(memory_space=pltpu.SMEM),
            pl.BlockSpec((s, LANES), lambda i: (i, 0)),
        ],
        out_specs=[
            pl.BlockSpec((T, s, LANES), lambda i: (0, i, 0)),
            pl.BlockSpec(memory_space=pltpu.SMEM),
            pl.BlockSpec(memory_space=pltpu.SMEM),
        ],
        out_shape=[
            jax.ShapeDtypeStruct((T, rows, LANES), jnp.float32),
            jax.ShapeDtypeStruct((nblk, T), jnp.int32),
            jax.ShapeDtypeStruct((1, 1), jnp.int32),
        ],
        compiler_params=pltpu.CompilerParams(
            dimension_semantics=("arbitrary",)),
    )(ksubs, rsubs, flat2d)

    spikes = pl.pallas_call(
        _make_pass_b(s, n_pix),
        grid=grid,
        in_specs=[
            pl.BlockSpec(memory_space=pltpu.SMEM),
            pl.BlockSpec(memory_space=pltpu.SMEM),
            pl.BlockSpec(memory_space=pltpu.SMEM),
            pl.BlockSpec((s, LANES), lambda i: (i, 0)),
            pl.BlockSpec((T, s, LANES), lambda i: (0, i, 0)),
        ],
        out_specs=pl.BlockSpec((T, s, LANES), lambda i: (0, i, 0)),
        out_shape=jax.ShapeDtypeStruct((T, rows, LANES), jnp.bool_),
        compiler_params=pltpu.CompilerParams(
            dimension_semantics=("arbitrary",)),
    )(rsubs, n_arr, m_arr, flat2d, ival_a)

    return spikes.reshape((T,) + tuple(shape))
